# baseline (speedup 1.0000x reference)
import functools

import jax
import jax.numpy as jnp
from jax import lax
from jax.experimental import pallas as pl
from jax.experimental.pallas import tpu as pltpu
from jax.experimental.pallas import tpu_sc as plsc

BATCH = 4096
MAX_LEN = 200
D_MODEL = 64
LANES = 16
NUM_CORES = 2
NUM_SUBCORES = 16
NUM_WORKERS = NUM_CORES * NUM_SUBCORES
BPW = BATCH // NUM_WORKERS
SPLIT = (104, 96)
ROW_BYTES = D_MODEL * 4


def kernel(x, token_table, pos_table):
    mesh = plsc.VectorSubcoreMesh(core_axis_name="c", subcore_axis_name="s")

    @functools.partial(
        pl.kernel,
        out_type=jax.ShapeDtypeStruct((BATCH, MAX_LEN, D_MODEL), jnp.float32),
        mesh=mesh,
        compiler_params=pltpu.CompilerParams(use_tc_tiling_on_sc=False),
        scratch_types=[
            pltpu.VMEM((BPW, MAX_LEN), jnp.int32),
            pltpu.VMEM((MAX_LEN, D_MODEL), jnp.float32),
            pltpu.VMEM((SPLIT[0], D_MODEL), jnp.float32),
            pltpu.VMEM((SPLIT[1], D_MODEL), jnp.float32),
            pltpu.VMEM((SPLIT[0], D_MODEL), jnp.float32),
            pltpu.VMEM((SPLIT[1], D_MODEL), jnp.float32),
        ] + [pltpu.SemaphoreType.DMA] * 8,
    )
    def sc_kernel(x_hbm, tok_hbm, pos_hbm, out_hbm, idx_v, pos_v,
                  r00, r01, r10, r11, *sems):
        rows = ((r00, r01), (r10, r11))
        gsem = (sems[0:2], sems[2:4])
        osem = (sems[4:6], sems[6:8])
        wid = lax.axis_index("s") * NUM_CORES + lax.axis_index("c")
        b0 = wid * BPW
        pltpu.sync_copy(x_hbm.at[pl.ds(b0, BPW)], idx_v)
        pltpu.sync_copy(pos_hbm, pos_v)

        def gather_op(r, a, c):
            l0 = 0 if c == 0 else SPLIT[0]
            return pltpu.make_async_copy(
                tok_hbm.at[idx_v.at[r, pl.ds(l0, SPLIT[c])]],
                rows[a][c], gsem[a][c])

        def put_op(r, a, c):
            l0 = 0 if c == 0 else SPLIT[0]
            return pltpu.make_async_copy(
                rows[a][c], out_hbm.at[b0 + r, pl.ds(l0, SPLIT[c])],
                osem[a][c])

        for a in range(2):
            for c in range(2):
                gather_op(a, a, c).start()

        @pl.loop(0, BPW, step=2)
        def _(g):
            for a in range(2):
                for c in range(2):
                    r = g + a
                    l0 = 0 if c == 0 else SPLIT[0]
                    gather_op(r, a, c).wait()
                    buf = rows[a][c]

                    @pl.loop(0, SPLIT[c])
                    def _(i):
                        for j in range(D_MODEL // LANES):
                            sl = pl.ds(j * LANES, LANES)
                            plsc.addupdate(buf.at[i, sl], pos_v[l0 + i, sl])

                    put_op(r, a, c).start()
            for a in range(2):
                for c in range(2):
                    r = g + a
                    put_op(r, a, c).wait()

                    @pl.when(r + 2 < BPW)
                    def _():
                        gather_op(r + 2, a, c).start()

    return sc_kernel(x.astype(jnp.int32), token_table, pos_table)
